# Initial kernel scaffold; baseline (speedup 1.0000x reference)
import jax
import jax.numpy as jnp
from jax.experimental import pallas as pl


def kernel(feat, params, edge_index):
    raise NotImplementedError("write your pallas kernel here")


# speedup vs baseline: 6.5046x; 6.5046x over previous
import functools
import math

import jax
import jax.numpy as jnp
from jax import lax
from jax.experimental import pallas as pl
from jax.experimental.pallas import tpu as pltpu
from jax.experimental.pallas import tpu_sc as plsc

N = 10000
E = 320000
D = 128
OUT = 40
EPS = 1e-5
BNS = 1.0 / math.sqrt(1.0 + EPS)

NC, NS = 2, 16
NW = NC * NS
EPW = E // NW
CB = 128
NFULL = EPW // CB
TAIL = EPW - NFULL * CB
NPAD = 10240
RPT = NPAD // NS
QCH = RPT // CB


def _make_agg(with_deg: bool):
    out_type = [jax.ShapeDtypeStruct((NC, NPAD, D), jnp.float32)]
    if with_deg:
        out_type.append(jax.ShapeDtypeStruct((NC, NPAD), jnp.float32))

    scratch = [
        pltpu.VMEM((CB,), jnp.int32),
        pltpu.VMEM((CB,), jnp.int32),
        pltpu.VMEM((TAIL,), jnp.int32),
        pltpu.VMEM((TAIL,), jnp.int32),
        pltpu.VMEM((CB, D), jnp.float32),
        pltpu.VMEM((TAIL, D), jnp.float32),
        pltpu.VMEM((CB,), jnp.float32),
        pltpu.VMEM((CB,), jnp.float32),
        pltpu.VMEM_SHARED((NPAD, D), jnp.float32),
        pltpu.VMEM_SHARED((NPAD,), jnp.float32),
        pltpu.SemaphoreType.DMA,
    ]

    def body(h_hbm, src_hbm, dst_hbm, z_hbm, one_hbm, *refs):
        if with_deg:
            aggp, degp = refs[0], refs[1]
            rest = refs[2:]
        else:
            aggp, degp = refs[0], None
            rest = refs[1:]
        (src_v, dst_v, s16, d16, rows_v, rows16, ones_v, dv,
         acc_sh, deg_sh, sem) = rest

        c = lax.axis_index("c")
        s = lax.axis_index("s")
        w = s * NC + c

        pltpu.sync_copy(z_hbm, rows_v)
        pltpu.sync_copy(one_hbm, ones_v)

        def zero_q(q, carry):
            off = s * RPT + q * CB
            pltpu.sync_copy(rows_v, acc_sh.at[pl.ds(off, CB)])
            if with_deg:
                pltpu.sync_copy(rows_v.at[0], deg_sh.at[pl.ds(off, CB)])
            return carry

        lax.fori_loop(0, QCH, zero_q, 0)
        plsc.subcore_barrier()

        base0 = w * EPW

        def edge_chunk(j, carry):
            b = base0 + j * CB
            pltpu.sync_copy(src_hbm.at[pl.ds(b, CB)], src_v)
            pltpu.sync_copy(dst_hbm.at[pl.ds(b, CB)], dst_v)
            pltpu.async_copy(h_hbm.at[src_v], rows_v, sem).wait()
            pltpu.sync_copy(rows_v, acc_sh.at[dst_v], add=True)
            if with_deg:
                pltpu.sync_copy(ones_v, deg_sh.at[dst_v], add=True)
            return carry

        lax.fori_loop(0, NFULL, edge_chunk, 0)

        bt = base0 + NFULL * CB
        pltpu.sync_copy(src_hbm.at[pl.ds(bt, TAIL)], s16)
        pltpu.sync_copy(dst_hbm.at[pl.ds(bt, TAIL)], d16)
        pltpu.async_copy(h_hbm.at[s16], rows16, sem).wait()
        pltpu.sync_copy(rows16, acc_sh.at[d16], add=True)
        if with_deg:
            pltpu.sync_copy(ones_v.at[pl.ds(0, TAIL)], deg_sh.at[d16], add=True)

        plsc.subcore_barrier()

        def wb_q(q, carry):
            off = s * RPT + q * CB
            pltpu.sync_copy(acc_sh.at[pl.ds(off, CB)], rows_v)
            pltpu.sync_copy(rows_v, aggp.at[c, pl.ds(off, CB)])
            if with_deg:
                pltpu.sync_copy(deg_sh.at[pl.ds(off, CB)], dv)
                pltpu.sync_copy(dv, degp.at[c, pl.ds(off, CB)])
            return carry

        lax.fori_loop(0, QCH, wb_q, 0)

    mesh = plsc.VectorSubcoreMesh(
        core_axis_name="c", subcore_axis_name="s",
        num_cores=NC, num_subcores=NS)
    return pl.kernel(body, out_type=out_type, mesh=mesh,
                     scratch_types=scratch)


_agg_deg = _make_agg(True)
_agg = _make_agg(False)

BR = 1000
GRID = N // BR


def _conv_body(h_ref, p_ref, d_ref, ws_ref, wn_ref, g_ref, b_ref, o_ref):
    dsum = jnp.maximum(d_ref[0] + d_ref[1], 1.0)
    agg = (p_ref[0] + p_ref[1]) / dsum
    rst = (jnp.dot(h_ref[...], ws_ref[...], preferred_element_type=jnp.float32)
           + jnp.dot(agg, wn_ref[...], preferred_element_type=jnp.float32))
    y = rst * (g_ref[0] * BNS) + b_ref[0]
    o_ref[...] = jnp.maximum(y, 0.0)


def _final_body(h_ref, p_ref, d_ref, ws_ref, wn_ref, g_ref, b_ref,
                w0_ref, b0_ref, g0_ref, be0_ref, w1_ref, b1_ref, o_ref):
    dsum = jnp.maximum(d_ref[0] + d_ref[1], 1.0)
    agg = (p_ref[0] + p_ref[1]) / dsum
    rst = (jnp.dot(h_ref[...], ws_ref[...], preferred_element_type=jnp.float32)
           + jnp.dot(agg, wn_ref[...], preferred_element_type=jnp.float32))
    h3 = jnp.maximum(rst * (g_ref[0] * BNS) + b_ref[0], 0.0)
    t = jnp.dot(h3, w0_ref[...], preferred_element_type=jnp.float32) + b0_ref[0]
    t = jnp.maximum(t * (g0_ref[0] * BNS) + be0_ref[0], 0.0)
    o_ref[...] = (jnp.dot(t, w1_ref[...], preferred_element_type=jnp.float32)
                  + b1_ref[0])


_ROWS = pl.BlockSpec((BR, D), lambda i: (i, 0))
_PART = pl.BlockSpec((NC, BR, D), lambda i: (0, i, 0))
_DEG = pl.BlockSpec((NC, BR, 1), lambda i: (0, i, 0))
_MAT = pl.BlockSpec((D, D), lambda i: (0, 0))
_VEC = pl.BlockSpec((1, D), lambda i: (0, 0))

_conv_tc = pl.pallas_call(
    _conv_body,
    grid=(GRID,),
    in_specs=[_ROWS, _PART, _DEG, _MAT, _MAT, _VEC, _VEC],
    out_specs=_ROWS,
    out_shape=jax.ShapeDtypeStruct((N, D), jnp.float32),
)

_final_tc = pl.pallas_call(
    _final_body,
    grid=(GRID,),
    in_specs=[_ROWS, _PART, _DEG, _MAT, _MAT, _VEC, _VEC,
              _MAT, _VEC, _VEC, _VEC,
              pl.BlockSpec((D, OUT), lambda i: (0, 0)),
              pl.BlockSpec((1, OUT), lambda i: (0, 0))],
    out_specs=pl.BlockSpec((BR, OUT), lambda i: (i, 0)),
    out_shape=jax.ShapeDtypeStruct((N, OUT), jnp.float32),
)


def kernel(feat, params, edge_index):
    src = edge_index[0]
    dst = edge_index[1]
    zeros = jnp.zeros((CB, D), jnp.float32)
    ones = jnp.ones((CB,), jnp.float32)

    convs = params["convs"]
    c0, c1 = params["cls"][0], params["cls"][1]
    row = lambda v: v.reshape(1, -1)

    h = feat
    degp3 = None
    for i in range(len(convs)):
        p = convs[i]
        if i == 0:
            aggp, degp = _agg_deg(h, src, dst, zeros, ones)
            degp3 = degp[:, :, None]
        else:
            (aggp,) = _agg(h, src, dst, zeros, ones)
        if i < len(convs) - 1:
            h = _conv_tc(h, aggp, degp3, p["W_self"], p["W_neigh"],
                         row(p["gamma"]), row(p["beta"]))
        else:
            h = _final_tc(h, aggp, degp3, p["W_self"], p["W_neigh"],
                          row(p["gamma"]), row(p["beta"]),
                          c0["W"], row(c0["b"]), row(c0["gamma"]),
                          row(c0["beta"]), c1["W"], row(c1["b"]))
    return h


# speedup vs baseline: 9.9250x; 1.5258x over previous
import functools
import math

import jax
import jax.numpy as jnp
from jax import lax
from jax.experimental import pallas as pl
from jax.experimental.pallas import tpu as pltpu
from jax.experimental.pallas import tpu_sc as plsc

N = 10000
E = 320000
D = 128
OUT = 40
EPS = 1e-5
BNS = 1.0 / math.sqrt(1.0 + EPS)

NC, NS = 2, 16
NW = NC * NS
EPW = E // NW
CB = 128
NFULL = EPW // CB
TAIL = EPW - NFULL * CB
NPAD = 10240
RPT = NPAD // NS
QCH = RPT // CB


def _make_agg(with_deg: bool):
    out_type = [jax.ShapeDtypeStruct((NC, NPAD, D), jnp.float32)]
    if with_deg:
        out_type.append(jax.ShapeDtypeStruct((NC, NPAD), jnp.float32))

    scratch = [
        pltpu.VMEM((1, CB), jnp.int32),
        pltpu.VMEM((1, CB), jnp.int32),
        pltpu.VMEM((1, CB), jnp.int32),
        pltpu.VMEM((1, CB), jnp.int32),
        pltpu.VMEM((TAIL,), jnp.int32),
        pltpu.VMEM((TAIL,), jnp.int32),
        pltpu.VMEM((CB, D), jnp.float32),
        pltpu.VMEM((CB, D), jnp.float32),
        pltpu.VMEM((TAIL, D), jnp.float32),
        pltpu.VMEM((CB,), jnp.float32),
        pltpu.VMEM((CB,), jnp.float32),
        pltpu.VMEM_SHARED((NPAD, D), jnp.float32),
        pltpu.VMEM_SHARED((NPAD,), jnp.float32),
        pltpu.SemaphoreType.DMA,
    ]

    def body(h_hbm, src_hbm, dst_hbm, z_hbm, one_hbm, *refs):
        if with_deg:
            aggp, degp = refs[0], refs[1]
            rest = refs[2:]
        else:
            aggp, degp = refs[0], None
            rest = refs[1:]
        (srcA, srcB, dstA, dstB, s16, d16, rowsA, rowsB, rows16,
         ones_v, dv, acc_sh, deg_sh, sem) = rest

        c = lax.axis_index("c")
        s = lax.axis_index("s")
        w = s * NC + c

        pltpu.sync_copy(z_hbm, rowsA)
        pltpu.sync_copy(one_hbm, ones_v)

        def zero_q(q, carry):
            off = s * RPT + q * CB
            pltpu.sync_copy(rowsA, acc_sh.at[pl.ds(off, CB)])
            if with_deg:
                pltpu.sync_copy(rowsA.at[0], deg_sh.at[pl.ds(off, CB)])
            return carry

        lax.fori_loop(0, QCH, zero_q, 0)
        plsc.subcore_barrier()

        base0 = w * EPW

        def load_idx(g, sbuf, dbuf):
            b = base0 + g * CB
            pltpu.sync_copy(src_hbm.at[pl.ds(b, CB)], sbuf.at[0])
            pltpu.sync_copy(dst_hbm.at[pl.ds(b, CB)], dbuf.at[0])

        def fire(sbuf, rbuf):
            pltpu.async_copy(h_hbm.at[sbuf.at[0]], rbuf, sem)

        def drain(sbuf, rbuf):
            pltpu.make_async_copy(h_hbm.at[sbuf.at[0]], rbuf, sem).wait()

        def scat(dbuf, rbuf):
            pltpu.sync_copy(rbuf, acc_sh.at[dbuf.at[0]], add=True)
            if with_deg:
                pltpu.sync_copy(ones_v, deg_sh.at[dbuf.at[0]], add=True)

        load_idx(0, srcA, dstA)
        fire(srcA, rowsA)
        NIT = NFULL // 2

        def pipe(it, carry):
            g = it * 2
            load_idx(g + 1, srcB, dstB)
            fire(srcB, rowsB)
            drain(srcA, rowsA)
            scat(dstA, rowsA)

            @pl.when(it < NIT - 1)
            def _():
                load_idx(g + 2, srcA, dstA)
                fire(srcA, rowsA)

            drain(srcB, rowsB)
            scat(dstB, rowsB)
            return carry

        lax.fori_loop(0, NIT, pipe, 0)

        bt = base0 + NFULL * CB
        pltpu.sync_copy(src_hbm.at[pl.ds(bt, TAIL)], s16)
        pltpu.sync_copy(dst_hbm.at[pl.ds(bt, TAIL)], d16)
        pltpu.async_copy(h_hbm.at[s16], rows16, sem).wait()
        pltpu.sync_copy(rows16, acc_sh.at[d16], add=True)
        if with_deg:
            pltpu.sync_copy(ones_v.at[pl.ds(0, TAIL)], deg_sh.at[d16], add=True)

        plsc.subcore_barrier()

        def wb_q(q, carry):
            off = s * RPT + q * CB
            pltpu.sync_copy(acc_sh.at[pl.ds(off, CB)], rowsA)
            pltpu.sync_copy(rowsA, aggp.at[c, pl.ds(off, CB)])
            if with_deg:
                pltpu.sync_copy(deg_sh.at[pl.ds(off, CB)], dv)
                pltpu.sync_copy(dv, degp.at[c, pl.ds(off, CB)])
            return carry

        lax.fori_loop(0, QCH, wb_q, 0)

    mesh = plsc.VectorSubcoreMesh(
        core_axis_name="c", subcore_axis_name="s",
        num_cores=NC, num_subcores=NS)
    return pl.kernel(body, out_type=out_type, mesh=mesh,
                     scratch_types=scratch)


_agg_deg = _make_agg(True)
_agg = _make_agg(False)

BR = 1000
GRID = N // BR


def _conv_body(h_ref, p_ref, d_ref, ws_ref, wn_ref, g_ref, b_ref, o_ref):
    dsum = jnp.maximum(d_ref[0] + d_ref[1], 1.0)
    agg = (p_ref[0] + p_ref[1]) / dsum
    rst = (jnp.dot(h_ref[...], ws_ref[...], preferred_element_type=jnp.float32)
           + jnp.dot(agg, wn_ref[...], preferred_element_type=jnp.float32))
    y = rst * (g_ref[0] * BNS) + b_ref[0]
    o_ref[...] = jnp.maximum(y, 0.0)


def _final_body(h_ref, p_ref, d_ref, ws_ref, wn_ref, g_ref, b_ref,
                w0_ref, b0_ref, g0_ref, be0_ref, w1_ref, b1_ref, o_ref):
    dsum = jnp.maximum(d_ref[0] + d_ref[1], 1.0)
    agg = (p_ref[0] + p_ref[1]) / dsum
    rst = (jnp.dot(h_ref[...], ws_ref[...], preferred_element_type=jnp.float32)
           + jnp.dot(agg, wn_ref[...], preferred_element_type=jnp.float32))
    h3 = jnp.maximum(rst * (g_ref[0] * BNS) + b_ref[0], 0.0)
    t = jnp.dot(h3, w0_ref[...], preferred_element_type=jnp.float32) + b0_ref[0]
    t = jnp.maximum(t * (g0_ref[0] * BNS) + be0_ref[0], 0.0)
    o_ref[...] = (jnp.dot(t, w1_ref[...], preferred_element_type=jnp.float32)
                  + b1_ref[0])


_ROWS = pl.BlockSpec((BR, D), lambda i: (i, 0))
_PART = pl.BlockSpec((NC, BR, D), lambda i: (0, i, 0))
_DEG = pl.BlockSpec((NC, BR, 1), lambda i: (0, i, 0))
_MAT = pl.BlockSpec((D, D), lambda i: (0, 0))
_VEC = pl.BlockSpec((1, D), lambda i: (0, 0))

_conv_tc = pl.pallas_call(
    _conv_body,
    grid=(GRID,),
    in_specs=[_ROWS, _PART, _DEG, _MAT, _MAT, _VEC, _VEC],
    out_specs=_ROWS,
    out_shape=jax.ShapeDtypeStruct((N, D), jnp.float32),
)

_final_tc = pl.pallas_call(
    _final_body,
    grid=(GRID,),
    in_specs=[_ROWS, _PART, _DEG, _MAT, _MAT, _VEC, _VEC,
              _MAT, _VEC, _VEC, _VEC,
              pl.BlockSpec((D, OUT), lambda i: (0, 0)),
              pl.BlockSpec((1, OUT), lambda i: (0, 0))],
    out_specs=pl.BlockSpec((BR, OUT), lambda i: (i, 0)),
    out_shape=jax.ShapeDtypeStruct((N, OUT), jnp.float32),
)


def kernel(feat, params, edge_index):
    src = edge_index[0]
    dst = edge_index[1]
    zeros = jnp.zeros((CB, D), jnp.float32)
    ones = jnp.ones((CB,), jnp.float32)

    convs = params["convs"]
    c0, c1 = params["cls"][0], params["cls"][1]
    row = lambda v: v.reshape(1, -1)

    h = feat
    degp3 = None
    for i in range(len(convs)):
        p = convs[i]
        if i == 0:
            aggp, degp = _agg_deg(h, src, dst, zeros, ones)
            degp3 = degp[:, :, None]
        else:
            (aggp,) = _agg(h, src, dst, zeros, ones)
        if i < len(convs) - 1:
            h = _conv_tc(h, aggp, degp3, p["W_self"], p["W_neigh"],
                         row(p["gamma"]), row(p["beta"]))
        else:
            h = _final_tc(h, aggp, degp3, p["W_self"], p["W_neigh"],
                          row(p["gamma"]), row(p["beta"]),
                          c0["W"], row(c0["b"]), row(c0["gamma"]),
                          row(c0["beta"]), c1["W"], row(c1["b"]))
    return h
